# baseline (speedup 1.0000x reference)
import functools

import jax
import jax.numpy as jnp
import numpy as np
from jax import lax
from jax.experimental import pallas as pl
from jax.experimental.pallas import tpu as pltpu
from jax.experimental.pallas import tpu_sc as plsc

BATCH = 16384
HALF = BATCH // 2
EMBED_DIM = 128
_CHUNK = 128
_NBUF = 2


def _convert_chunk(fbuf, pbuf, b):
    def body(m, carry):
        rin = pl.multiple_of(m * 8, 8)
        for dr in range(8):
            for g in range(EMBED_DIM // 16):
                a = fbuf[b, 0, rin + dr, pl.ds(g * 16, 16)]
                c = fbuf[b, 1, rin + dr, pl.ds(g * 16, 16)]
                lo = ((a + 0x8000) >> 16) & np.int32(0xFFFF)
                hi = (c + 0x8000) & np.int32(-65536)
                pbuf[b, rin + dr, pl.ds(g * 16, 16)] = lo | hi
        return carry
    lax.fori_loop(0, _CHUNK // 8, body, 0)


def _gather_tec_body(nc, bpw, uidx, iidx, utab, itab, uep_out, iep_out,
                     uidx_v, iidx_v, fbuf, pbuf, *sems):
    wid = lax.axis_index("s") * nc + lax.axis_index("c")
    base = wid * bpw
    nck = bpw // _CHUNK
    pltpu.sync_copy(uidx.at[pl.ds(base, bpw)], uidx_v.at[pl.ds(0, bpw)])
    pltpu.sync_copy(uidx.at[pl.ds(HALF + base, bpw)],
                    uidx_v.at[pl.ds(bpw, bpw)])
    pltpu.sync_copy(iidx.at[pl.ds(base, bpw)], iidx_v.at[pl.ds(0, bpw)])
    pltpu.sync_copy(iidx.at[pl.ds(HALF + base, bpw)],
                    iidx_v.at[pl.ds(bpw, bpw)])
    gsems = sems[:_NBUF]
    ssems = sems[_NBUF:]
    tasks = ([(uidx_v, utab, uep_out, j) for j in range(nck)]
             + [(iidx_v, itab, iep_out, j) for j in range(nck)])
    nt = len(tasks)
    gathers = [None] * _NBUF
    stores = [None] * _NBUF

    def fire(t):
        iv, tab, _, j = tasks[t]
        b = t % _NBUF
        if stores[b] is not None:
            stores[b].wait()
        g_lo = pltpu.async_copy(
            tab.at[iv.at[pl.ds(j * _CHUNK, _CHUNK)]], fbuf.at[b, 0], gsems[b])
        g_hi = pltpu.async_copy(
            tab.at[iv.at[pl.ds(bpw + j * _CHUNK, _CHUNK)]], fbuf.at[b, 1],
            gsems[b])
        gathers[b] = (g_lo, g_hi)

    def finish(t):
        b = t % _NBUF
        _, _, dst, j = tasks[t]
        gathers[b][0].wait()
        gathers[b][1].wait()
        _convert_chunk(fbuf, pbuf, b)
        stores[b] = pltpu.async_copy(
            pbuf.at[b], dst.at[pl.ds(base + j * _CHUNK, _CHUNK)], ssems[b])

    for t in range(nt + 1):
        if t < nt:
            fire(t)
        if t >= 1:
            finish(t - 1)
    for s in stores:
        if s is not None:
            s.wait()


def _sc_gather(user_indices, item_indices, user_emb_i32, item_emb_i32):
    info = plsc.get_sparse_core_info()
    nc, ns = info.num_cores, info.num_subcores
    nw = nc * ns
    bpw = HALF // nw
    mesh = plsc.VectorSubcoreMesh(core_axis_name="c", subcore_axis_name="s")
    k = pl.kernel(
        functools.partial(_gather_tec_body, nc, bpw),
        mesh=mesh,
        out_type=[
            jax.ShapeDtypeStruct((HALF, EMBED_DIM), jnp.int32),
            jax.ShapeDtypeStruct((HALF, EMBED_DIM), jnp.int32),
        ],
        scratch_types=[
            pltpu.VMEM((2 * bpw,), jnp.int32),
            pltpu.VMEM((2 * bpw,), jnp.int32),
            pltpu.VMEM((_NBUF, 2, _CHUNK, EMBED_DIM), jnp.int32),
            pltpu.VMEM((_NBUF, _CHUNK, EMBED_DIM), jnp.int32),
        ] + [pltpu.SemaphoreType.DMA] * (2 * _NBUF),
        compiler_params=pltpu.CompilerParams(use_tc_tiling_on_sc=False),
    )
    return k(user_indices, item_indices, user_emb_i32, item_emb_i32)


def _unpack(xi32):
    lo = lax.bitcast_convert_type(
        jnp.left_shift(xi32, 16), jnp.float32).astype(jnp.bfloat16)
    hi = lax.bitcast_convert_type(
        xi32 & np.int32(-65536), jnp.float32).astype(jnp.bfloat16)
    return lo, hi


def _mlp_body(uep, iep, w1u, w1i, b1, w2, b2, w3, b3, wo, bo,
              out_lo, out_hi):
    ue_lo, ue_hi = _unpack(uep[...])
    ie_lo, ie_hi = _unpack(iep[...])

    def stream(xu, xi, out):
        x = jnp.dot(xu, w1u[...], preferred_element_type=jnp.float32)
        x = x + jnp.dot(xi, w1i[...], preferred_element_type=jnp.float32)
        x = jnp.maximum(x + b1[...], 0.0).astype(jnp.bfloat16)
        x = jnp.maximum(jnp.dot(x, w2[...], preferred_element_type=jnp.float32) + b2[...], 0.0)
        x = jnp.maximum(jnp.dot(x, w3[...], preferred_element_type=jnp.float32) + b3[...], 0.0)
        y = lax.dot_general(wo[...], x, (((1,), (1,)), ((), ())),
                            preferred_element_type=jnp.float32)
        out[...] = y.reshape(out.shape) + bo[0, 0]

    stream(ue_lo, ie_lo, out_lo)
    stream(ue_hi, ie_hi, out_hi)


def _tc_mlp(uep, iep, w1u_t, w1i_t, b1, w2_t, b2, w3_t, b3, wo, bo):
    blk = 2048
    grid = HALF // blk
    full = lambda shape: pl.BlockSpec(shape, lambda i: (0, 0))
    out_lo, out_hi = pl.pallas_call(
        _mlp_body,
        grid=(grid,),
        in_specs=[
            pl.BlockSpec((blk, EMBED_DIM), lambda i: (i, 0)),
            pl.BlockSpec((blk, EMBED_DIM), lambda i: (i, 0)),
            full(w1u_t.shape),
            full(w1i_t.shape),
            full(b1.shape),
            full(w2_t.shape),
            full(b2.shape),
            full(w3_t.shape),
            full(b3.shape),
            full(wo.shape),
            full(bo.shape),
        ],
        out_specs=[
            pl.BlockSpec((blk // 128, 128), lambda i: (i, 0)),
            pl.BlockSpec((blk // 128, 128), lambda i: (i, 0)),
        ],
        out_shape=[
            jax.ShapeDtypeStruct((HALF // 128, 128), jnp.float32),
            jax.ShapeDtypeStruct((HALF // 128, 128), jnp.float32),
        ],
    )(uep, iep, w1u_t, w1i_t, b1, w2_t, b2, w3_t, b3, wo, bo)
    return jnp.concatenate([out_lo.reshape(HALF), out_hi.reshape(HALF)])


def kernel(user_indices, item_indices, user_emb, item_emb,
           W1, b1, W2, b2, W3, b3, Wo, bo):
    user_indices = user_indices.astype(jnp.int32)
    item_indices = item_indices.astype(jnp.int32)
    uep, iep = _sc_gather(
        user_indices, item_indices,
        lax.bitcast_convert_type(user_emb, jnp.int32),
        lax.bitcast_convert_type(item_emb, jnp.int32))
    w1u_t = W1[:, :EMBED_DIM].T.astype(jnp.bfloat16)
    w1i_t = W1[:, EMBED_DIM:].T.astype(jnp.bfloat16)
    return _tc_mlp(
        uep, iep,
        w1u_t, w1i_t, b1.reshape(1, -1),
        W2.T.astype(jnp.bfloat16), b2.reshape(1, -1),
        W3.T, b3.reshape(1, -1),
        Wo, bo.reshape(1, 1),
    )


# speedup vs baseline: 1.0070x; 1.0070x over previous
import functools

import jax
import jax.numpy as jnp
import numpy as np
from jax import lax
from jax.experimental import pallas as pl
from jax.experimental.pallas import tpu as pltpu
from jax.experimental.pallas import tpu_sc as plsc

BATCH = 16384
HALF = BATCH // 2
EMBED_DIM = 128
_CHUNK = 128
_NBUF = 2


def _convert_chunk(fbuf, pbuf, b):
    @plsc.parallel_loop(0, _CHUNK, step=8)
    def _(rin):
        rin = pl.multiple_of(rin, 8)
        for dr in range(8):
            for g in range(EMBED_DIM // 16):
                a = fbuf[b, 0, rin + dr, pl.ds(g * 16, 16)]
                c = fbuf[b, 1, rin + dr, pl.ds(g * 16, 16)]
                lo = ((a + 0x8000) >> 16) & np.int32(0xFFFF)
                hi = (c + 0x8000) & np.int32(-65536)
                pbuf[b, rin + dr, pl.ds(g * 16, 16)] = lo | hi


def _gather_tec_body(nc, bpw, uidx, iidx, utab, itab, uep_out, iep_out,
                     uidx_v, iidx_v, fbuf, pbuf, *sems):
    wid = lax.axis_index("s") * nc + lax.axis_index("c")
    base = wid * bpw
    nck = bpw // _CHUNK
    pltpu.sync_copy(uidx.at[pl.ds(base, bpw)], uidx_v.at[pl.ds(0, bpw)])
    pltpu.sync_copy(uidx.at[pl.ds(HALF + base, bpw)],
                    uidx_v.at[pl.ds(bpw, bpw)])
    pltpu.sync_copy(iidx.at[pl.ds(base, bpw)], iidx_v.at[pl.ds(0, bpw)])
    pltpu.sync_copy(iidx.at[pl.ds(HALF + base, bpw)],
                    iidx_v.at[pl.ds(bpw, bpw)])
    gsems = sems[:_NBUF]
    ssems = sems[_NBUF:]
    tasks = ([(uidx_v, utab, uep_out, j) for j in range(nck)]
             + [(iidx_v, itab, iep_out, j) for j in range(nck)])
    nt = len(tasks)
    gathers = [None] * _NBUF
    stores = [None] * _NBUF

    def fire(t):
        iv, tab, _, j = tasks[t]
        b = t % _NBUF
        if stores[b] is not None:
            stores[b].wait()
        g_lo = pltpu.async_copy(
            tab.at[iv.at[pl.ds(j * _CHUNK, _CHUNK)]], fbuf.at[b, 0], gsems[b])
        g_hi = pltpu.async_copy(
            tab.at[iv.at[pl.ds(bpw + j * _CHUNK, _CHUNK)]], fbuf.at[b, 1],
            gsems[b])
        gathers[b] = (g_lo, g_hi)

    def finish(t):
        b = t % _NBUF
        _, _, dst, j = tasks[t]
        gathers[b][0].wait()
        gathers[b][1].wait()
        _convert_chunk(fbuf, pbuf, b)
        stores[b] = pltpu.async_copy(
            pbuf.at[b], dst.at[pl.ds(base + j * _CHUNK, _CHUNK)], ssems[b])

    for t in range(nt + 1):
        if t < nt:
            fire(t)
        if t >= 1:
            finish(t - 1)
    for s in stores:
        if s is not None:
            s.wait()


def _sc_gather(user_indices, item_indices, user_emb_i32, item_emb_i32):
    info = plsc.get_sparse_core_info()
    nc, ns = info.num_cores, info.num_subcores
    nw = nc * ns
    bpw = HALF // nw
    mesh = plsc.VectorSubcoreMesh(core_axis_name="c", subcore_axis_name="s")
    k = pl.kernel(
        functools.partial(_gather_tec_body, nc, bpw),
        mesh=mesh,
        out_type=[
            jax.ShapeDtypeStruct((HALF, EMBED_DIM), jnp.int32),
            jax.ShapeDtypeStruct((HALF, EMBED_DIM), jnp.int32),
        ],
        scratch_types=[
            pltpu.VMEM((2 * bpw,), jnp.int32),
            pltpu.VMEM((2 * bpw,), jnp.int32),
            pltpu.VMEM((_NBUF, 2, _CHUNK, EMBED_DIM), jnp.int32),
            pltpu.VMEM((_NBUF, _CHUNK, EMBED_DIM), jnp.int32),
        ] + [pltpu.SemaphoreType.DMA] * (2 * _NBUF),
        compiler_params=pltpu.CompilerParams(use_tc_tiling_on_sc=False),
    )
    return k(user_indices, item_indices, user_emb_i32, item_emb_i32)


def _unpack(xi32):
    lo = lax.bitcast_convert_type(
        jnp.left_shift(xi32, 16), jnp.float32).astype(jnp.bfloat16)
    hi = lax.bitcast_convert_type(
        xi32 & np.int32(-65536), jnp.float32).astype(jnp.bfloat16)
    return lo, hi


def _mlp_body(uep, iep, w1u, w1i, b1, w2, b2, w3, b3, wo, bo,
              out_lo, out_hi):
    ue_lo, ue_hi = _unpack(uep[...])
    ie_lo, ie_hi = _unpack(iep[...])

    def stream(xu, xi, out):
        x = jnp.dot(xu, w1u[...], preferred_element_type=jnp.float32)
        x = x + jnp.dot(xi, w1i[...], preferred_element_type=jnp.float32)
        x = jnp.maximum(x + b1[...], 0.0).astype(jnp.bfloat16)
        x = jnp.maximum(jnp.dot(x, w2[...], preferred_element_type=jnp.float32) + b2[...], 0.0)
        x = jnp.maximum(jnp.dot(x, w3[...], preferred_element_type=jnp.float32) + b3[...], 0.0)
        y = lax.dot_general(wo[...], x, (((1,), (1,)), ((), ())),
                            preferred_element_type=jnp.float32)
        out[...] = y.reshape(out.shape) + bo[0, 0]

    stream(ue_lo, ie_lo, out_lo)
    stream(ue_hi, ie_hi, out_hi)


def _tc_mlp(uep, iep, w1u_t, w1i_t, b1, w2_t, b2, w3_t, b3, wo, bo):
    blk = 2048
    grid = HALF // blk
    full = lambda shape: pl.BlockSpec(shape, lambda i: (0, 0))
    out_lo, out_hi = pl.pallas_call(
        _mlp_body,
        grid=(grid,),
        in_specs=[
            pl.BlockSpec((blk, EMBED_DIM), lambda i: (i, 0)),
            pl.BlockSpec((blk, EMBED_DIM), lambda i: (i, 0)),
            full(w1u_t.shape),
            full(w1i_t.shape),
            full(b1.shape),
            full(w2_t.shape),
            full(b2.shape),
            full(w3_t.shape),
            full(b3.shape),
            full(wo.shape),
            full(bo.shape),
        ],
        out_specs=[
            pl.BlockSpec((blk // 128, 128), lambda i: (i, 0)),
            pl.BlockSpec((blk // 128, 128), lambda i: (i, 0)),
        ],
        out_shape=[
            jax.ShapeDtypeStruct((HALF // 128, 128), jnp.float32),
            jax.ShapeDtypeStruct((HALF // 128, 128), jnp.float32),
        ],
    )(uep, iep, w1u_t, w1i_t, b1, w2_t, b2, w3_t, b3, wo, bo)
    return jnp.concatenate([out_lo.reshape(HALF), out_hi.reshape(HALF)])


def kernel(user_indices, item_indices, user_emb, item_emb,
           W1, b1, W2, b2, W3, b3, Wo, bo):
    user_indices = user_indices.astype(jnp.int32)
    item_indices = item_indices.astype(jnp.int32)
    uep, iep = _sc_gather(
        user_indices, item_indices,
        lax.bitcast_convert_type(user_emb, jnp.int32),
        lax.bitcast_convert_type(item_emb, jnp.int32))
    w1u_t = W1[:, :EMBED_DIM].T.astype(jnp.bfloat16)
    w1i_t = W1[:, EMBED_DIM:].T.astype(jnp.bfloat16)
    return _tc_mlp(
        uep, iep,
        w1u_t, w1i_t, b1.reshape(1, -1),
        W2.T.astype(jnp.bfloat16), b2.reshape(1, -1),
        W3.T, b3.reshape(1, -1),
        Wo, bo.reshape(1, 1),
    )


# speedup vs baseline: 2.1718x; 2.1567x over previous
import functools

import jax
import jax.numpy as jnp
from jax import lax
from jax.experimental import pallas as pl
from jax.experimental.pallas import tpu as pltpu
from jax.experimental.pallas import tpu_sc as plsc

BATCH = 16384
EMBED_DIM = 128
_CHUNK = 128


_NBUF = 4


def _gather_tec_body(nc, bpw, uidx, iidx, utab, itab, ue_out, ie_out,
                     uidx_v, iidx_v, buf, *sems):
    wid = lax.axis_index("s") * nc + lax.axis_index("c")
    base = wid * bpw
    nck = bpw // _CHUNK
    pltpu.sync_copy(uidx.at[pl.ds(base, bpw)], uidx_v)
    pltpu.sync_copy(iidx.at[pl.ds(base, bpw)], iidx_v)
    gsems = sems[:_NBUF]
    ssems = sems[_NBUF:]
    tasks = ([(uidx_v, utab, ue_out, j) for j in range(nck)]
             + [(iidx_v, itab, ie_out, j) for j in range(nck)])
    gathers = [None] * _NBUF
    stores = [None] * _NBUF

    def drain(t):
        b = t % _NBUF
        _, _, dst, j = tasks[t]
        gathers[b].wait()
        stores[b] = pltpu.async_copy(
            buf.at[b], dst.at[pl.ds(base + j * _CHUNK, _CHUNK)], ssems[b])

    for t, (iv, tab, dst, j) in enumerate(tasks):
        b = t % _NBUF
        if stores[b] is not None:
            stores[b].wait()
        gathers[b] = pltpu.async_copy(
            tab.at[iv.at[pl.ds(j * _CHUNK, _CHUNK)]], buf.at[b], gsems[b])
        if t >= _NBUF - 1:
            drain(t - _NBUF + 1)
    for t in range(len(tasks) - _NBUF + 1, len(tasks)):
        drain(t)
    for s in stores:
        if s is not None:
            s.wait()


def _sc_gather(user_indices, item_indices, user_emb, item_emb):
    info = plsc.get_sparse_core_info()
    nc, ns = info.num_cores, info.num_subcores
    nw = nc * ns
    bpw = BATCH // nw
    mesh = plsc.VectorSubcoreMesh(core_axis_name="c", subcore_axis_name="s")
    k = pl.kernel(
        functools.partial(_gather_tec_body, nc, bpw),
        mesh=mesh,
        out_type=[
            jax.ShapeDtypeStruct((BATCH, EMBED_DIM), jnp.float32),
            jax.ShapeDtypeStruct((BATCH, EMBED_DIM), jnp.float32),
        ],
        scratch_types=[
            pltpu.VMEM((bpw,), jnp.int32),
            pltpu.VMEM((bpw,), jnp.int32),
            pltpu.VMEM((_NBUF, _CHUNK, EMBED_DIM), jnp.float32),
        ] + [pltpu.SemaphoreType.DMA] * (2 * _NBUF),
    )
    return k(user_indices, item_indices, user_emb, item_emb)


def _mlp_body(ue, ie, w1u, w1i, b1, w2, b2, w3, b3, wo, bo, out):
    xu = ue[...].astype(jnp.bfloat16)
    xi = ie[...].astype(jnp.bfloat16)
    x = jnp.dot(xu, w1u[...], preferred_element_type=jnp.float32)
    x = x + jnp.dot(xi, w1i[...], preferred_element_type=jnp.float32)
    x = jnp.maximum(x + b1[...], 0.0).astype(jnp.bfloat16)
    x = jnp.maximum(jnp.dot(x, w2[...], preferred_element_type=jnp.float32) + b2[...], 0.0)
    x = jnp.maximum(jnp.dot(x, w3[...], preferred_element_type=jnp.float32) + b3[...], 0.0)
    y = lax.dot_general(wo[...], x, (((1,), (1,)), ((), ())),
                        preferred_element_type=jnp.float32)
    out[...] = y.reshape(out.shape) + bo[0, 0]


def _tc_mlp(ue, ie, w1u_t, w1i_t, b1, w2_t, b2, w3_t, b3, wo, bo):
    blk = 2048
    grid = BATCH // blk
    full = lambda shape: pl.BlockSpec(shape, lambda i: (0, 0))
    return pl.pallas_call(
        _mlp_body,
        grid=(grid,),
        in_specs=[
            pl.BlockSpec((blk, EMBED_DIM), lambda i: (i, 0)),
            pl.BlockSpec((blk, EMBED_DIM), lambda i: (i, 0)),
            full(w1u_t.shape),
            full(w1i_t.shape),
            full(b1.shape),
            full(w2_t.shape),
            full(b2.shape),
            full(w3_t.shape),
            full(b3.shape),
            full(wo.shape),
            full(bo.shape),
        ],
        out_specs=pl.BlockSpec((blk // 128, 128), lambda i: (i, 0)),
        out_shape=jax.ShapeDtypeStruct((BATCH // 128, 128), jnp.float32),
    )(ue, ie, w1u_t, w1i_t, b1, w2_t, b2, w3_t, b3, wo, bo).reshape(BATCH)


def kernel(user_indices, item_indices, user_emb, item_emb,
           W1, b1, W2, b2, W3, b3, Wo, bo):
    user_indices = user_indices.astype(jnp.int32)
    item_indices = item_indices.astype(jnp.int32)
    ue, ie = _sc_gather(user_indices, item_indices, user_emb, item_emb)
    w1u_t = W1[:, :EMBED_DIM].T.astype(jnp.bfloat16)
    w1i_t = W1[:, EMBED_DIM:].T.astype(jnp.bfloat16)
    return _tc_mlp(
        ue, ie,
        w1u_t, w1i_t, b1.reshape(1, -1),
        W2.T.astype(jnp.bfloat16), b2.reshape(1, -1),
        W3.T, b3.reshape(1, -1),
        Wo, bo.reshape(1, 1),
    )


# speedup vs baseline: 2.3393x; 1.0771x over previous
import functools

import jax
import jax.numpy as jnp
from jax import lax
from jax.experimental import pallas as pl
from jax.experimental.pallas import tpu as pltpu
from jax.experimental.pallas import tpu_sc as plsc

BATCH = 16384
EMBED_DIM = 128
_CHUNK = 128


_NBUF = 4


def _gather_tec_body(nc, bpw, uidx, iidx, utab, itab, cat_out,
                     uidx_v, iidx_v, buf, *sems):
    wid = lax.axis_index("s") * nc + lax.axis_index("c")
    base = wid * bpw
    nck = bpw // _CHUNK
    pltpu.sync_copy(uidx.at[pl.ds(base, bpw)], uidx_v)
    pltpu.sync_copy(iidx.at[pl.ds(base, bpw)], iidx_v)
    gsems = sems[:_NBUF]
    ssems = sems[_NBUF:]
    tasks = ([(uidx_v, utab, 0, j) for j in range(nck)]
             + [(iidx_v, itab, EMBED_DIM, j) for j in range(nck)])
    gathers = [None] * _NBUF
    stores = [None] * _NBUF

    def drain(t):
        b = t % _NBUF
        _, _, col0, j = tasks[t]
        gathers[b].wait()
        stores[b] = pltpu.async_copy(
            buf.at[b],
            cat_out.at[pl.ds(base + j * _CHUNK, _CHUNK),
                       pl.ds(col0, EMBED_DIM)],
            ssems[b])

    for t, (iv, tab, col0, j) in enumerate(tasks):
        b = t % _NBUF
        if stores[b] is not None:
            stores[b].wait()
        gathers[b] = pltpu.async_copy(
            tab.at[iv.at[pl.ds(j * _CHUNK, _CHUNK)]], buf.at[b], gsems[b])
        if t >= _NBUF - 1:
            drain(t - _NBUF + 1)
    for t in range(len(tasks) - _NBUF + 1, len(tasks)):
        drain(t)
    for s in stores:
        if s is not None:
            s.wait()


def _sc_gather(user_indices, item_indices, user_emb, item_emb):
    info = plsc.get_sparse_core_info()
    nc, ns = info.num_cores, info.num_subcores
    nw = nc * ns
    bpw = BATCH // nw
    mesh = plsc.VectorSubcoreMesh(core_axis_name="c", subcore_axis_name="s")
    k = pl.kernel(
        functools.partial(_gather_tec_body, nc, bpw),
        mesh=mesh,
        out_type=jax.ShapeDtypeStruct((BATCH, 2 * EMBED_DIM), jnp.float32),
        scratch_types=[
            pltpu.VMEM((bpw,), jnp.int32),
            pltpu.VMEM((bpw,), jnp.int32),
            pltpu.VMEM((_NBUF, _CHUNK, EMBED_DIM), jnp.float32),
        ] + [pltpu.SemaphoreType.DMA] * (2 * _NBUF),
    )
    return k(user_indices, item_indices, user_emb, item_emb)


def _mlp_body(xin, w1, b1, w2, b2, w3, b3, wo, bo, out):
    x = jnp.dot(xin[...].astype(jnp.bfloat16), w1[...],
                preferred_element_type=jnp.float32)
    x = jnp.maximum(x + b1[...], 0.0).astype(jnp.bfloat16)
    x = jnp.maximum(jnp.dot(x, w2[...], preferred_element_type=jnp.float32) + b2[...], 0.0)
    x = jnp.maximum(jnp.dot(x, w3[...], preferred_element_type=jnp.float32) + b3[...], 0.0)
    y = lax.dot_general(wo[...], x, (((1,), (1,)), ((), ())),
                        preferred_element_type=jnp.float32)
    out[...] = y.reshape(out.shape) + bo[0, 0]


def _tc_mlp(xcat, w1_t, b1, w2_t, b2, w3_t, b3, wo, bo):
    blk = 2048
    grid = BATCH // blk
    full = lambda shape: pl.BlockSpec(shape, lambda i: (0, 0))
    return pl.pallas_call(
        _mlp_body,
        grid=(grid,),
        in_specs=[
            pl.BlockSpec((blk, 2 * EMBED_DIM), lambda i: (i, 0)),
            full(w1_t.shape),
            full(b1.shape),
            full(w2_t.shape),
            full(b2.shape),
            full(w3_t.shape),
            full(b3.shape),
            full(wo.shape),
            full(bo.shape),
        ],
        out_specs=pl.BlockSpec((blk // 128, 128), lambda i: (i, 0)),
        out_shape=jax.ShapeDtypeStruct((BATCH // 128, 128), jnp.float32),
    )(xcat, w1_t, b1, w2_t, b2, w3_t, b3, wo, bo).reshape(BATCH)


def kernel(user_indices, item_indices, user_emb, item_emb,
           W1, b1, W2, b2, W3, b3, Wo, bo):
    user_indices = user_indices.astype(jnp.int32)
    item_indices = item_indices.astype(jnp.int32)
    xcat = _sc_gather(user_indices, item_indices, user_emb, item_emb)
    return _tc_mlp(
        xcat,
        W1.T.astype(jnp.bfloat16), b1.reshape(1, -1),
        W2.T.astype(jnp.bfloat16), b2.reshape(1, -1),
        W3.T, b3.reshape(1, -1),
        Wo, bo.reshape(1, 1),
    )


# speedup vs baseline: 2.3508x; 1.0049x over previous
import functools

import jax
import jax.numpy as jnp
from jax import lax
from jax.experimental import pallas as pl
from jax.experimental.pallas import tpu as pltpu
from jax.experimental.pallas import tpu_sc as plsc

BATCH = 16384
EMBED_DIM = 128
_CHUNK = 128


_NBUF = 4


def _gather_tec_body(nc, bpw, uidx, iidx, utab, itab, cat_out,
                     uidx_v, iidx_v, buf, *sems):
    wid = lax.axis_index("s") * nc + lax.axis_index("c")
    base = wid * bpw
    nck = bpw // _CHUNK
    pltpu.sync_copy(uidx.at[pl.ds(base, bpw)], uidx_v)
    pltpu.sync_copy(iidx.at[pl.ds(base, bpw)], iidx_v)
    gsems = sems[:_NBUF]
    ssems = sems[_NBUF:]
    tasks = ([(uidx_v, utab, 0, j) for j in range(nck)]
             + [(iidx_v, itab, EMBED_DIM, j) for j in range(nck)])
    gathers = [None] * _NBUF
    stores = [None] * _NBUF

    def drain(t):
        b = t % _NBUF
        _, _, col0, j = tasks[t]
        gathers[b].wait()
        stores[b] = pltpu.async_copy(
            buf.at[b],
            cat_out.at[pl.ds(base + j * _CHUNK, _CHUNK),
                       pl.ds(col0, EMBED_DIM)],
            ssems[b])

    for t, (iv, tab, col0, j) in enumerate(tasks):
        b = t % _NBUF
        if stores[b] is not None:
            stores[b].wait()
        gathers[b] = pltpu.async_copy(
            tab.at[iv.at[pl.ds(j * _CHUNK, _CHUNK)]], buf.at[b], gsems[b])
        if t >= _NBUF - 1:
            drain(t - _NBUF + 1)
    for t in range(len(tasks) - _NBUF + 1, len(tasks)):
        drain(t)
    for s in stores:
        if s is not None:
            s.wait()


def _sc_gather(user_indices, item_indices, user_emb, item_emb):
    info = plsc.get_sparse_core_info()
    nc, ns = info.num_cores, info.num_subcores
    nw = nc * ns
    bpw = BATCH // nw
    mesh = plsc.VectorSubcoreMesh(core_axis_name="c", subcore_axis_name="s")
    k = pl.kernel(
        functools.partial(_gather_tec_body, nc, bpw),
        mesh=mesh,
        out_type=jax.ShapeDtypeStruct((BATCH, 2 * EMBED_DIM), jnp.float32),
        scratch_types=[
            pltpu.VMEM((bpw,), jnp.int32),
            pltpu.VMEM((bpw,), jnp.int32),
            pltpu.VMEM((_NBUF, _CHUNK, EMBED_DIM), jnp.float32),
        ] + [pltpu.SemaphoreType.DMA] * (2 * _NBUF),
    )
    return k(user_indices, item_indices, user_emb, item_emb)


def _mlp_body(xin, w1, b1, w2, b2, w3, b3, wo, bo, out):
    x = jnp.dot(xin[...].astype(jnp.bfloat16), w1[...],
                preferred_element_type=jnp.float32)
    x = jnp.maximum(x + b1[...], 0.0).astype(jnp.bfloat16)
    x = jnp.maximum(jnp.dot(x, w2[...], preferred_element_type=jnp.float32) + b2[...], 0.0)
    x = jnp.maximum(jnp.dot(x, w3[...], preferred_element_type=jnp.float32) + b3[...], 0.0)
    y = lax.dot_general(wo[...], x, (((1,), (1,)), ((), ())),
                        preferred_element_type=jnp.float32)
    out[...] = y.reshape(out.shape) + bo[0, 0]


def _tc_mlp(xcat, w1_t, b1, w2_t, b2, w3_t, b3, wo, bo):
    blk = 4096
    grid = BATCH // blk
    full = lambda shape: pl.BlockSpec(shape, lambda i: (0, 0))
    return pl.pallas_call(
        _mlp_body,
        grid=(grid,),
        in_specs=[
            pl.BlockSpec((blk, 2 * EMBED_DIM), lambda i: (i, 0)),
            full(w1_t.shape),
            full(b1.shape),
            full(w2_t.shape),
            full(b2.shape),
            full(w3_t.shape),
            full(b3.shape),
            full(wo.shape),
            full(bo.shape),
        ],
        out_specs=pl.BlockSpec((blk // 128, 128), lambda i: (i, 0)),
        out_shape=jax.ShapeDtypeStruct((BATCH // 128, 128), jnp.float32),
    )(xcat, w1_t, b1, w2_t, b2, w3_t, b3, wo, bo).reshape(BATCH)


def kernel(user_indices, item_indices, user_emb, item_emb,
           W1, b1, W2, b2, W3, b3, Wo, bo):
    user_indices = user_indices.astype(jnp.int32)
    item_indices = item_indices.astype(jnp.int32)
    xcat = _sc_gather(user_indices, item_indices, user_emb, item_emb)
    return _tc_mlp(
        xcat,
        W1.T.astype(jnp.bfloat16), b1.reshape(1, -1),
        W2.T.astype(jnp.bfloat16), b2.reshape(1, -1),
        W3.T, b3.reshape(1, -1),
        Wo, bo.reshape(1, 1),
    )
